# baseline (speedup 1.0000x reference)
import functools

import jax
import jax.numpy as jnp
from jax import lax
from jax.experimental import pallas as pl
from jax.experimental.pallas import tpu as pltpu
from jax.experimental.pallas import tpu_sc as plsc

B = 16384
VOCAB = 1000
HID = 10
VPAD = 1024
TROWS = 16
NC = 2
NS = 16
NW = NC * NS
L = 16
BPW = B // NW
GROUPS = BPW // L
NPAR = 144


def _prep_body(emb_ref, w1_ref, out_ref):
    tb = jnp.dot(emb_ref[...], w1_ref[1:, :],
                 preferred_element_type=jnp.float32,
                 precision=lax.Precision.HIGHEST)
    tb = jnp.concatenate([tb, jnp.zeros((VOCAB, TROWS - HID), jnp.float32)],
                         axis=1)
    tbj = tb.T
    out_ref[...] = jnp.concatenate(
        [tbj, jnp.zeros((TROWS, VPAD - VOCAB), jnp.float32)], axis=1)


def _prep(emb, W1):
    return pl.pallas_call(
        _prep_body,
        out_shape=jax.ShapeDtypeStruct((TROWS, VPAD), jnp.float32),
    )(emb, W1)


def _sc_body(tb_hbm, par_hbm, ae_hbm, out_hbm,
             tb_sh, tb_v, par_v, ae_v, out_v):
    cid = lax.axis_index("c")
    sid = lax.axis_index("s")
    wid = sid * NC + cid
    base = wid * BPW

    @pl.when(sid == 0)
    def _stage():
        pltpu.sync_copy(tb_hbm, tb_sh)

    pltpu.sync_copy(par_hbm, par_v)
    pltpu.sync_copy(ae_hbm.at[pl.ds(base, BPW)], ae_v)
    plsc.subcore_barrier()
    pltpu.sync_copy(tb_sh, tb_v)

    GB = 2

    def group(g):
        offs = [(g + t) * L for t in range(GB)]
        packed = [ae_v[pl.ds(o, L)] for o in offs]
        edus = [jnp.right_shift(p, 7) for p in packed]
        gs = [[plsc.load_gather(tb_v, [jnp.full((L,), k, jnp.int32), edus[t]])
               for k in range(HID)] for t in range(GB)]
        ages = [(p & 127).astype(jnp.float32) for p in packed]
        acc = [[None] * HID for _ in range(GB)]
        for k in range(HID):
            w1k = par_v[k]
            b1k = par_v[131 + k]
            h1k = [jnp.maximum(ages[t] * w1k + gs[t][k] + b1k, 0.0)
                   for t in range(GB)]
            for j in range(HID):
                w2kj = par_v[10 + k * HID + j]
                for t in range(GB):
                    if k == 0:
                        acc[t][j] = h1k[t] * w2kj
                    else:
                        acc[t][j] = acc[t][j] + h1k[t] * w2kj
        o_out = [par_v[130] for _ in range(GB)]
        for j in range(HID):
            b2j = par_v[110 + j]
            w3j = par_v[120 + j]
            for t in range(GB):
                h2tj = jnp.maximum(acc[t][j] + b2j, 0.0)
                o_out[t] = o_out[t] + h2tj * w3j
        for t in range(GB):
            out_v[pl.ds(offs[t], L)] = 1.0 / (1.0 + jnp.exp(-o_out[t]))

    plsc.parallel_loop(0, GROUPS, GB, unroll=2)(group)
    pltpu.sync_copy(out_v, out_hbm.at[pl.ds(base, BPW)])


@functools.cache
def _sc_mlp():
    return functools.partial(
        pl.kernel,
        out_type=jax.ShapeDtypeStruct((B,), jnp.float32),
        mesh=plsc.VectorSubcoreMesh(core_axis_name="c", subcore_axis_name="s",
                                    num_cores=NC, num_subcores=NS),
        scratch_types=[
            pltpu.VMEM_SHARED((TROWS, VPAD), jnp.float32),
            pltpu.VMEM((TROWS, VPAD), jnp.float32),
            pltpu.VMEM((NPAR, L), jnp.float32),
            pltpu.VMEM((BPW,), jnp.int32),
            pltpu.VMEM((BPW,), jnp.float32),
        ],
        compiler_params=pltpu.CompilerParams(needs_layout_passes=False),
    )(_sc_body)


def kernel(age, education, emb, W1, b1, W2, b2, W3, b3):
    tbj = _prep(emb, W1)
    pars = jnp.concatenate([
        W1[0, :], W2.reshape(-1), b2, W3[:, 0], b3, b1,
        jnp.zeros((NPAR - 141,), jnp.float32),
    ])
    par2d = jnp.broadcast_to(pars[:, None], (NPAR, L))
    ae = (education.reshape(B) * 128 + age.reshape(B)).astype(jnp.int32)
    out = _sc_mlp()(tbj, par2d, ae)
    return out.reshape(B, 1)


# speedup vs baseline: 1.0075x; 1.0075x over previous
import functools

import jax
import jax.numpy as jnp
from jax import lax
from jax.experimental import pallas as pl
from jax.experimental.pallas import tpu as pltpu
from jax.experimental.pallas import tpu_sc as plsc

B = 16384
VOCAB = 1000
HID = 10
VPAD = 1024
TROWS = 16
NC = 2
NS = 16
NW = NC * NS
L = 16
BPW = B // NW
GROUPS = BPW // L
NPAR = 144


def _prep_body(emb_ref, w1_ref, out_ref):
    tbj = lax.dot_general(
        w1_ref[1:, :], emb_ref[...],
        dimension_numbers=(((0,), (1,)), ((), ())),
        preferred_element_type=jnp.float32,
        precision=lax.Precision.HIGHEST,
    )
    tbj = jnp.concatenate([tbj, jnp.zeros((TROWS - HID, VOCAB), jnp.float32)],
                          axis=0)
    out_ref[...] = jnp.concatenate(
        [tbj, jnp.zeros((TROWS, VPAD - VOCAB), jnp.float32)], axis=1)


def _prep(emb, W1):
    return pl.pallas_call(
        _prep_body,
        out_shape=jax.ShapeDtypeStruct((TROWS, VPAD), jnp.float32),
    )(emb, W1)


def _sc_body(tb_hbm, par_hbm, ae_hbm, out_hbm,
             tb_sh, tb_v, par_v, ae_v, out_v):
    cid = lax.axis_index("c")
    sid = lax.axis_index("s")
    wid = sid * NC + cid
    base = wid * BPW

    @pl.when(sid == 0)
    def _stage():
        pltpu.sync_copy(tb_hbm, tb_sh)

    pltpu.sync_copy(par_hbm, par_v)
    pltpu.sync_copy(ae_hbm.at[pl.ds(base, BPW)], ae_v)
    plsc.subcore_barrier()
    pltpu.sync_copy(tb_sh, tb_v)

    GB = 2

    def group(g):
        offs = [(g + t) * L for t in range(GB)]
        packed = [ae_v[pl.ds(o, L)] for o in offs]
        edus = [jnp.right_shift(p, 7) for p in packed]
        gs = [[plsc.load_gather(tb_v, [jnp.full((L,), k, jnp.int32), edus[t]])
               for k in range(HID)] for t in range(GB)]
        ages = [(p & 127).astype(jnp.float32) for p in packed]
        acc = [[None] * HID for _ in range(GB)]
        for k in range(HID):
            w1k = par_v[k]
            b1k = par_v[131 + k]
            h1k = [jnp.maximum(ages[t] * w1k + gs[t][k] + b1k, 0.0)
                   for t in range(GB)]
            for j in range(HID):
                w2kj = par_v[10 + k * HID + j]
                for t in range(GB):
                    if k == 0:
                        acc[t][j] = h1k[t] * w2kj
                    else:
                        acc[t][j] = acc[t][j] + h1k[t] * w2kj
        o_out = [par_v[130] for _ in range(GB)]
        for j in range(HID):
            b2j = par_v[110 + j]
            w3j = par_v[120 + j]
            for t in range(GB):
                h2tj = jnp.maximum(acc[t][j] + b2j, 0.0)
                o_out[t] = o_out[t] + h2tj * w3j
        for t in range(GB):
            out_v[pl.ds(offs[t], L)] = 1.0 / (1.0 + jnp.exp(-o_out[t]))

    plsc.parallel_loop(0, GROUPS, GB, unroll=4)(group)
    pltpu.sync_copy(out_v, out_hbm.at[pl.ds(base, BPW)])


@functools.cache
def _sc_mlp():
    return functools.partial(
        pl.kernel,
        out_type=jax.ShapeDtypeStruct((B,), jnp.float32),
        mesh=plsc.VectorSubcoreMesh(core_axis_name="c", subcore_axis_name="s",
                                    num_cores=NC, num_subcores=NS),
        scratch_types=[
            pltpu.VMEM_SHARED((TROWS, VPAD), jnp.float32),
            pltpu.VMEM((TROWS, VPAD), jnp.float32),
            pltpu.VMEM((NPAR, L), jnp.float32),
            pltpu.VMEM((BPW,), jnp.int32),
            pltpu.VMEM((BPW,), jnp.float32),
        ],
        compiler_params=pltpu.CompilerParams(needs_layout_passes=False),
    )(_sc_body)


def kernel(age, education, emb, W1, b1, W2, b2, W3, b3):
    tbj = _prep(emb, W1)
    pars = jnp.concatenate([
        W1[0, :], W2.reshape(-1), b2, W3[:, 0], b3, b1,
        jnp.zeros((NPAR - 141,), jnp.float32),
    ])
    par2d = jnp.broadcast_to(pars[:, None], (NPAR, L))
    ae = (education.reshape(B) * 128 + age.reshape(B)).astype(jnp.int32)
    out = _sc_mlp()(tbj, par2d, ae)
    return out.reshape(B, 1)
